# baseline (speedup 1.0000x reference)
import functools

import jax
import jax.numpy as jnp
from jax import lax
from jax.experimental import pallas as pl
from jax.experimental.pallas import tpu as pltpu
from jax.experimental.pallas import tpu_sc as plsc

B = 1024
S = 200
H = 128
TOK = B * S

NC = 2
NS = 16
NW = NC * NS
CH = 128

_MESH = plsc.VectorSubcoreMesh(core_axis_name="c", subcore_axis_name="s")


def _sc_gather(table, idx3):
    nch = idx3.shape[1]
    cpw = nch * CH
    tok = NW * cpw

    nbuf = 5
    assert nch >= nbuf and (nch - nbuf) % nbuf == 0

    @functools.partial(
        pl.kernel,
        mesh=_MESH,
        out_type=jax.ShapeDtypeStruct((tok, H), jnp.float32),
        scratch_types=(
            [pltpu.VMEM((nch, CH), jnp.int32)]
            + [pltpu.VMEM((CH, H), jnp.float32) for _ in range(nbuf)]
            + [pltpu.SemaphoreType.DMA, pltpu.SemaphoreType.DMA]
        ),
    )
    def k(table_hbm, idx_hbm, out_hbm, idx_v, *rest):
        rows = rest[:nbuf]
        sg, sw = rest[nbuf], rest[nbuf + 1]
        NCH = nch
        wid = lax.axis_index("s") * NC + lax.axis_index("c")
        base = wid * cpw
        pltpu.sync_copy(idx_hbm.at[wid], idx_v)

        def out_at(j):
            return out_hbm.at[pl.ds(base + j * CH, CH)]

        cg = [pltpu.async_copy(table_hbm.at[idx_v.at[b]], rows[b], sg)
              for b in range(nbuf)]
        for b in range(nbuf):
            cg[b].wait()
            pltpu.async_copy(rows[b], out_at(b), sw)

        @pl.loop(nbuf, NCH, step=nbuf)
        def _(j):
            c = []
            for b in range(nbuf):
                pltpu.make_async_copy(rows[b], out_at(j - nbuf + b), sw).wait()
                c.append(pltpu.async_copy(
                    table_hbm.at[idx_v.at[j + b]], rows[b], sg))
            for b in range(nbuf):
                c[b].wait()
                pltpu.async_copy(rows[b], out_at(j + b), sw)

        for b in range(nbuf):
            pltpu.make_async_copy(rows[b], out_at(NCH - nbuf + b), sw).wait()

    return k(table, idx3)


def _ln_body(g_ref, tt_ref, pc0_ref, pc1_ref, gam_ref, bet_ref, o_ref):
    x = g_ref[...] + jnp.where(tt_ref[...] != 0., pc1_ref[...], pc0_ref[...])
    mu = jnp.mean(x, axis=-1, keepdims=True)
    msq = jnp.mean(x * x, axis=-1, keepdims=True)
    inv = lax.rsqrt(msq - mu * mu + 1e-5)
    o_ref[...] = (((x - mu) * inv) * gam_ref[...]) + bet_ref[...]


_RB = 16
_TBLK = _RB * S


def _ln_call(g2, tt2, pc0t, pc1t, gam2, bet2):
    ntok = g2.shape[0]
    grid = (ntok // _TBLK,)
    return pl.pallas_call(
        _ln_body,
        grid=grid,
        in_specs=[
            pl.BlockSpec((_TBLK, H), lambda i: (i, 0)),
            pl.BlockSpec((_TBLK, 1), lambda i: (i, 0)),
            pl.BlockSpec((_TBLK, H), lambda i: (0, 0)),
            pl.BlockSpec((_TBLK, H), lambda i: (0, 0)),
            pl.BlockSpec((1, H), lambda i: (0, 0)),
            pl.BlockSpec((1, H), lambda i: (0, 0)),
        ],
        out_specs=pl.BlockSpec((_TBLK, H), lambda i: (i, 0)),
        out_shape=jax.ShapeDtypeStruct((ntok, H), jnp.float32),
    )(g2, tt2, pc0t, pc1t, gam2, bet2)


_NSPLIT = 1


def kernel(input_ids, token_type_ids, token_embedding, pos_embedding,
           type_embedding, ln_gamma, ln_beta):
    pc0t = jnp.tile(pos_embedding[:S] + type_embedding[0], (_RB, 1))
    pc1t = jnp.tile(pos_embedding[:S] + type_embedding[1], (_RB, 1))
    gam2 = ln_gamma.reshape(1, H)
    bet2 = ln_beta.reshape(1, H)
    idx3 = input_ids.astype(jnp.int32).reshape(NW, TOK // (NW * CH), CH)
    tt2 = token_type_ids.astype(jnp.float32).reshape(TOK, 1)
    gathered = _sc_gather(token_embedding, idx3)
    out2 = _ln_call(gathered, tt2, pc0t, pc1t, gam2, bet2)
    return out2.reshape(B, S, H)


# speedup vs baseline: 1.0741x; 1.0741x over previous
import functools

import jax
import jax.numpy as jnp
from jax import lax
from jax.experimental import pallas as pl
from jax.experimental.pallas import tpu as pltpu
from jax.experimental.pallas import tpu_sc as plsc

B = 1024
S = 200
H = 128
TOK = B * S

NC = 2
NS = 16
NW = NC * NS
CH = 128

_MESH = plsc.VectorSubcoreMesh(core_axis_name="c", subcore_axis_name="s")


def _sc_gather(table, idx3):
    nch = idx3.shape[1]
    cpw = nch * CH
    tok = NW * cpw

    nbuf = 5
    assert nch >= nbuf and (nch - nbuf) % nbuf == 0

    @functools.partial(
        pl.kernel,
        mesh=_MESH,
        out_type=jax.ShapeDtypeStruct((tok, H), jnp.float32),
        scratch_types=(
            [pltpu.VMEM((nch, CH), jnp.int32)]
            + [pltpu.VMEM((CH, H), jnp.float32) for _ in range(nbuf)]
            + [pltpu.SemaphoreType.DMA, pltpu.SemaphoreType.DMA]
        ),
    )
    def k(table_hbm, idx_hbm, out_hbm, idx_v, *rest):
        rows = rest[:nbuf]
        sg, sw = rest[nbuf], rest[nbuf + 1]
        NCH = nch
        wid = lax.axis_index("s") * NC + lax.axis_index("c")
        base = wid * cpw
        pltpu.sync_copy(idx_hbm.at[wid], idx_v)

        def out_at(j):
            return out_hbm.at[pl.ds(base + j * CH, CH)]

        cg = [pltpu.async_copy(table_hbm.at[idx_v.at[b]], rows[b], sg)
              for b in range(nbuf)]
        for b in range(nbuf):
            cg[b].wait()
            pltpu.async_copy(rows[b], out_at(b), sw)

        @pl.loop(nbuf, NCH, step=nbuf)
        def _(j):
            c = []
            for b in range(nbuf):
                pltpu.make_async_copy(rows[b], out_at(j - nbuf + b), sw).wait()
                c.append(pltpu.async_copy(
                    table_hbm.at[idx_v.at[j + b]], rows[b], sg))
            for b in range(nbuf):
                c[b].wait()
                pltpu.async_copy(rows[b], out_at(j + b), sw)

        for b in range(nbuf):
            pltpu.make_async_copy(rows[b], out_at(NCH - nbuf + b), sw).wait()

    return k(table, idx3)


def _ln_body(g_ref, tt_ref, pc0_ref, pc1_ref, gam_ref, bet_ref, jm_ref,
             o_ref):
    x = g_ref[...] + jnp.where(tt_ref[...] != 0., pc1_ref[...], pc0_ref[...])
    xb = x.astype(jnp.bfloat16)
    jm = jm_ref[...]
    mu = jnp.dot(xb, jm, preferred_element_type=jnp.float32)
    msq = jnp.dot(xb * xb, jm, preferred_element_type=jnp.float32)
    inv = lax.rsqrt(msq - mu * mu + 1e-5)
    o_ref[...] = (((x - mu) * inv) * gam_ref[...]) + bet_ref[...]


_RB = 16
_TBLK = _RB * S


def _ln_call(g2, tt2, pc0t, pc1t, gam2, bet2, jm):
    ntok = g2.shape[0]
    grid = (ntok // _TBLK,)
    return pl.pallas_call(
        _ln_body,
        grid=grid,
        in_specs=[
            pl.BlockSpec((_TBLK, H), lambda i: (i, 0)),
            pl.BlockSpec((_TBLK, 1), lambda i: (i, 0)),
            pl.BlockSpec((_TBLK, H), lambda i: (0, 0)),
            pl.BlockSpec((_TBLK, H), lambda i: (0, 0)),
            pl.BlockSpec((1, H), lambda i: (0, 0)),
            pl.BlockSpec((1, H), lambda i: (0, 0)),
            pl.BlockSpec((H, H), lambda i: (0, 0)),
        ],
        out_specs=pl.BlockSpec((_TBLK, H), lambda i: (i, 0)),
        out_shape=jax.ShapeDtypeStruct((ntok, H), jnp.float32),
    )(g2, tt2, pc0t, pc1t, gam2, bet2, jm)


_NSPLIT = 1


def kernel(input_ids, token_type_ids, token_embedding, pos_embedding,
           type_embedding, ln_gamma, ln_beta):
    pc0t = jnp.tile(pos_embedding[:S] + type_embedding[0], (_RB, 1))
    pc1t = jnp.tile(pos_embedding[:S] + type_embedding[1], (_RB, 1))
    gam2 = ln_gamma.reshape(1, H)
    bet2 = ln_beta.reshape(1, H)
    idx3 = input_ids.astype(jnp.int32).reshape(NW, TOK // (NW * CH), CH)
    tt2 = token_type_ids.astype(jnp.float32).reshape(TOK, 1)
    jm = jnp.full((H, H), 1.0 / H, dtype=jnp.bfloat16)
    gathered = _sc_gather(token_embedding, idx3)
    out2 = _ln_call(gathered, tt2, pc0t, pc1t, gam2, bet2, jm)
    return out2.reshape(B, S, H)


# speedup vs baseline: 1.3417x; 1.2491x over previous
import functools

import jax
import jax.numpy as jnp
from jax import lax
from jax.experimental import pallas as pl
from jax.experimental.pallas import tpu as pltpu
from jax.experimental.pallas import tpu_sc as plsc

B = 1024
S = 200
H = 128
TOK = B * S

NC = 2
NS = 16
NW = NC * NS
CH = 128

_MESH = plsc.VectorSubcoreMesh(core_axis_name="c", subcore_axis_name="s")


def _sc_gather(table, idx3):
    nch = idx3.shape[1]
    cpw = nch * CH
    tok = NW * cpw

    nbuf = 5
    assert nch >= nbuf and (nch - nbuf) % nbuf == 0

    @functools.partial(
        pl.kernel,
        mesh=_MESH,
        out_type=jax.ShapeDtypeStruct((tok, H), jnp.float32),
        scratch_types=(
            [pltpu.VMEM((nch, CH), jnp.int32)]
            + [pltpu.VMEM((CH, H), jnp.float32) for _ in range(nbuf)]
            + [pltpu.SemaphoreType.DMA, pltpu.SemaphoreType.DMA]
        ),
    )
    def k(table_hbm, idx_hbm, out_hbm, idx_v, *rest):
        rows = rest[:nbuf]
        sg, sw = rest[nbuf], rest[nbuf + 1]
        NCH = nch
        wid = lax.axis_index("s") * NC + lax.axis_index("c")
        base = wid * cpw
        pltpu.sync_copy(idx_hbm.at[wid], idx_v)

        def out_at(j):
            return out_hbm.at[pl.ds(base + j * CH, CH)]

        cg = [pltpu.async_copy(table_hbm.at[idx_v.at[b]], rows[b], sg)
              for b in range(nbuf)]
        for b in range(nbuf):
            cg[b].wait()
            pltpu.async_copy(rows[b], out_at(b), sw)

        @pl.loop(nbuf, NCH, step=nbuf)
        def _(j):
            c = []
            for b in range(nbuf):
                pltpu.make_async_copy(rows[b], out_at(j - nbuf + b), sw).wait()
                c.append(pltpu.async_copy(
                    table_hbm.at[idx_v.at[j + b]], rows[b], sg))
            for b in range(nbuf):
                c[b].wait()
                pltpu.async_copy(rows[b], out_at(j + b), sw)

        for b in range(nbuf):
            pltpu.make_async_copy(rows[b], out_at(NCH - nbuf + b), sw).wait()

    return k(table, idx3)


def _ln_body(g_ref, tt_ref, pc0_ref, dt_ref, gam_ref, bet_ref, jm_ref,
             o_ref):
    ones_row = jnp.ones((1, H), dtype=jnp.bfloat16)
    ttm = lax.dot_general(tt_ref[...], ones_row,
                          (((0,), (0,)), ((), ())),
                          preferred_element_type=jnp.float32)
    x = g_ref[...] + pc0_ref[...] + ttm * dt_ref[...]
    xb = x.astype(jnp.bfloat16)
    jm = jm_ref[...]
    mu = jnp.dot(xb, jm, preferred_element_type=jnp.float32)
    msq = jnp.dot(xb * xb, jm, preferred_element_type=jnp.float32)
    inv = lax.rsqrt(msq - mu * mu + 1e-5)
    o_ref[...] = (((x - mu) * inv) * gam_ref[...]) + bet_ref[...]


_RB = 16
_TBLK = _RB * S


def _ln_call(g2, ttrow, pc0t, dt2, gam2, bet2, jm):
    ntok = g2.shape[0]
    grid = (ntok // _TBLK,)
    return pl.pallas_call(
        _ln_body,
        grid=grid,
        in_specs=[
            pl.BlockSpec((_TBLK, H), lambda i: (i, 0)),
            pl.BlockSpec((1, _TBLK), lambda i: (0, i)),
            pl.BlockSpec((_TBLK, H), lambda i: (0, 0)),
            pl.BlockSpec((1, H), lambda i: (0, 0)),
            pl.BlockSpec((1, H), lambda i: (0, 0)),
            pl.BlockSpec((1, H), lambda i: (0, 0)),
            pl.BlockSpec((H, H), lambda i: (0, 0)),
        ],
        out_specs=pl.BlockSpec((_TBLK, H), lambda i: (i, 0)),
        out_shape=jax.ShapeDtypeStruct((ntok, H), jnp.float32),
    )(g2, ttrow, pc0t, dt2, gam2, bet2, jm)


_NSPLIT = 1


def kernel(input_ids, token_type_ids, token_embedding, pos_embedding,
           type_embedding, ln_gamma, ln_beta):
    pc0t = jnp.tile(pos_embedding[:S] + type_embedding[0], (_RB, 1))
    dt2 = (type_embedding[1] - type_embedding[0]).reshape(1, H)
    gam2 = ln_gamma.reshape(1, H)
    bet2 = ln_beta.reshape(1, H)
    idx3 = input_ids.astype(jnp.int32).reshape(NW, TOK // (NW * CH), CH)
    ttrow = token_type_ids.astype(jnp.bfloat16).reshape(1, TOK)
    jm = jnp.full((H, H), 1.0 / H, dtype=jnp.bfloat16)
    gathered = _sc_gather(token_embedding, idx3)
    out2 = _ln_call(gathered, ttrow, pc0t, dt2, gam2, bet2, jm)
    return out2.reshape(B, S, H)
